# Initial kernel scaffold; baseline (speedup 1.0000x reference)
import jax
import jax.numpy as jnp
from jax.experimental import pallas as pl


def kernel(h, edge_index, W_structure, W_task):
    raise NotImplementedError("write your pallas kernel here")


# speedup vs baseline: 7.3984x; 7.3984x over previous
import jax
import jax.numpy as jnp
from jax import lax
from jax.experimental import pallas as pl
from jax.experimental.pallas import tpu as pltpu
from jax.experimental.pallas import tpu_sc as plsc

N_NODES = 10000
N_EDGES = 320000
D = 128
CENTER_NUM = 16
N_CLASSES = 40

N_PAD = 10240
CHUNK = 128
NUM_CHUNKS = N_EDGES // CHUNK
NUM_WORKERS = 32
MAX_CHUNKS_PER_TILE = (NUM_CHUNKS + NUM_WORKERS - 1) // NUM_WORKERS
ROWS_PER_TILE = N_PAD // 16
LOOP_ROW = N_NODES


def _sc_aggregate_body(h_hbm, ei_hbm, part_out, cnt_out,
                       acc_sh, src_v, dst_v, rows_v, zrow_v,
                       eqbuf_v, loopidx_v, sem):
    c = lax.axis_index("c")
    s = lax.axis_index("s")
    w = c * 16 + s

    zero16 = jnp.zeros((16,), jnp.float32)
    one16 = jnp.full((16,), 1.0, jnp.float32)

    def fill_const(i, carry):
        for q in range(D // 16):
            zrow_v[i, pl.ds(q * 16, 16)] = zero16
        return carry
    lax.fori_loop(0, 16, fill_const, 0)

    loopidx_v[...] = lax.iota(jnp.int32, 16) + LOOP_ROW

    base_row = s * ROWS_PER_TILE
    for q in range(ROWS_PER_TILE // 16):
        pltpu.sync_copy(zrow_v, acc_sh.at[pl.ds(base_row + q * 16, 16)])

    plsc.subcore_barrier()

    def chunk_body(j, eq_acc):
        cid = w + NUM_WORKERS * j
        valid = cid < NUM_CHUNKS

        @pl.when(valid)
        def _():
            base = cid * CHUNK
            pltpu.sync_copy(ei_hbm.at[0, pl.ds(base, CHUNK)], src_v)
            pltpu.sync_copy(ei_hbm.at[1, pl.ds(base, CHUNK)], dst_v)
            pltpu.async_copy(h_hbm.at[src_v], rows_v, sem).wait()
            pltpu.sync_copy(rows_v, acc_sh.at[dst_v], add=True)

        inc = zero16
        for q in range(CHUNK // 16):
            sv = src_v[pl.ds(q * 16, 16)]
            dv = dst_v[pl.ds(q * 16, 16)]
            inc = inc + jnp.where(sv == dv, 1.0, 0.0).astype(jnp.float32)
        return eq_acc + jnp.where(valid, inc, 0.0)

    eq = lax.fori_loop(0, MAX_CHUNKS_PER_TILE, chunk_body, zero16)

    plsc.subcore_barrier()

    pltpu.sync_copy(acc_sh.at[pl.ds(base_row, ROWS_PER_TILE)],
                    part_out.at[c, pl.ds(base_row, ROWS_PER_TILE)])
    for q in range(ROWS_PER_TILE // 16):
        pltpu.sync_copy(zrow_v, acc_sh.at[pl.ds(base_row + q * 16, 16)])

    def fill_ones(i, carry):
        for q in range(D // 16):
            rows_v[i, pl.ds(q * 16, 16)] = one16
        return carry
    lax.fori_loop(0, CHUNK, fill_ones, 0)

    plsc.subcore_barrier()

    def count_body(j, carry):
        cid = w + NUM_WORKERS * j

        @pl.when(cid < NUM_CHUNKS)
        def _():
            pltpu.sync_copy(ei_hbm.at[1, pl.ds(cid * CHUNK, CHUNK)], dst_v)
            pltpu.sync_copy(rows_v, acc_sh.at[dst_v], add=True)
        return carry
    lax.fori_loop(0, MAX_CHUNKS_PER_TILE, count_body, 0)

    def fill_eq(i, carry):
        eqbuf_v[i, pl.ds(0, 16)] = eq
        for q in range(1, D // 16):
            eqbuf_v[i, pl.ds(q * 16, 16)] = zero16
        return carry
    lax.fori_loop(0, 16, fill_eq, 0)
    pltpu.sync_copy(eqbuf_v, acc_sh.at[loopidx_v], add=True)

    plsc.subcore_barrier()

    pltpu.sync_copy(acc_sh.at[pl.ds(base_row, ROWS_PER_TILE)],
                    cnt_out.at[c, pl.ds(base_row, ROWS_PER_TILE)])


def _sc_aggregate(h, edge_index):
    mesh = plsc.VectorSubcoreMesh(core_axis_name="c", subcore_axis_name="s")
    return pl.kernel(
        _sc_aggregate_body,
        out_type=[
            jax.ShapeDtypeStruct((2, N_PAD, D), jnp.float32),
            jax.ShapeDtypeStruct((2, N_PAD, D), jnp.float32),
        ],
        mesh=mesh,
        scratch_types=[
            pltpu.VMEM_SHARED((N_PAD, D), jnp.float32),
            pltpu.VMEM((CHUNK,), jnp.int32),
            pltpu.VMEM((CHUNK,), jnp.int32),
            pltpu.VMEM((CHUNK, D), jnp.float32),
            pltpu.VMEM((16, D), jnp.float32),
            pltpu.VMEM((16, D), jnp.float32),
            pltpu.VMEM((16,), jnp.int32),
            pltpu.SemaphoreType.DMA,
        ],
    )(h, edge_index)


def _tc_dense_body(part_ref, cnt_ref, loop_ref, h_ref, ws_ref, wt_ref, out_ref):
    psum = part_ref[0] + part_ref[1]
    cnt = (cnt_ref[0] + cnt_ref[1])[:, 0:1]
    loop_total = jnp.sum(loop_ref[0] + loop_ref[1])
    loop_w = jnp.where(loop_total > 0.0, 0.0, 1.0)

    hm = (psum + loop_w * h_ref[...]) / jnp.maximum(cnt + loop_w, 1.0)

    logits = lax.dot_general(hm, ws_ref[...], (((1,), (1,)), ((), ())),
                             preferred_element_type=jnp.float32)
    maxv = jnp.max(logits, axis=1, keepdims=True)
    iota = lax.broadcasted_iota(jnp.int32, logits.shape, 1)
    idx = jnp.min(jnp.where(logits == maxv, iota, CENTER_NUM),
                  axis=1, keepdims=True)

    allout = lax.dot_general(hm, wt_ref[...], (((1,), (1,)), ((), ())),
                             preferred_element_type=jnp.float32)
    acc = jnp.zeros((out_ref.shape[0], N_CLASSES), jnp.float32)
    for k in range(CENTER_NUM):
        acc = acc + jnp.where(idx == k,
                              allout[:, k * N_CLASSES:(k + 1) * N_CLASSES],
                              0.0)
    out_ref[...] = acc


def _tc_dense(partial, cnt, h_pad, W_structure, Wt_flat):
    B = 256
    grid = (N_PAD // B,)
    return pl.pallas_call(
        _tc_dense_body,
        grid=grid,
        in_specs=[
            pl.BlockSpec((2, B, D), lambda i: (0, i, 0)),
            pl.BlockSpec((2, B, D), lambda i: (0, i, 0)),
            pl.BlockSpec((2, 16, D), lambda i: (0, LOOP_ROW // 16, 0)),
            pl.BlockSpec((B, D), lambda i: (i, 0)),
            pl.BlockSpec((CENTER_NUM, D), lambda i: (0, 0)),
            pl.BlockSpec((CENTER_NUM * N_CLASSES, D), lambda i: (0, 0)),
        ],
        out_specs=pl.BlockSpec((B, N_CLASSES), lambda i: (i, 0)),
        out_shape=jax.ShapeDtypeStruct((N_PAD, N_CLASSES), jnp.float32),
    )(partial, cnt, cnt, h_pad, W_structure, Wt_flat)


def kernel(h, edge_index, W_structure, W_task):
    partial, cnt = _sc_aggregate(h, edge_index)
    h_pad = jnp.pad(h, ((0, N_PAD - N_NODES), (0, 0)))
    Wt_flat = W_task.reshape(CENTER_NUM * N_CLASSES, D)
    out = _tc_dense(partial, cnt, h_pad, W_structure, Wt_flat)
    return out[:N_NODES]
